# Initial kernel scaffold; baseline (speedup 1.0000x reference)
import jax
import jax.numpy as jnp
from jax.experimental import pallas as pl


def kernel(x, edge_attr, edge_paths, edge_vector):
    raise NotImplementedError("write your pallas kernel here")


# speedup vs baseline: 42.0781x; 42.0781x over previous
import functools

import jax
import jax.numpy as jnp
from jax import lax
from jax.experimental import pallas as pl
from jax.experimental.pallas import tpu as pltpu
from jax.experimental.pallas import tpu_sc as plsc

_NC = 2
_NS = 16
_LANES = 16


def _w_table_kernel(vec_ref, attr_ref, out_ref):
    out_ref[...] = lax.dot_general(
        vec_ref[...], attr_ref[...],
        (((1,), (1,)), ((), ())),
        preferred_element_type=jnp.float32,
    )


def _make_sc_gather(n, e, l, chunk_elems, n_chunks):
    nw = _NC * _NS
    mesh = plsc.VectorSubcoreMesh(core_axis_name="c", subcore_axis_name="s")
    groups = chunk_elems // _LANES

    @functools.partial(
        pl.kernel,
        out_type=jax.ShapeDtypeStruct((n * n,), jnp.float32),
        mesh=mesh,
        compiler_params=pltpu.CompilerParams(needs_layout_passes=False),
        scratch_types=[
            pltpu.VMEM((l * e,), jnp.float32),
            pltpu.VMEM((chunk_elems * l,), jnp.int32),
            pltpu.VMEM((chunk_elems,), jnp.float32),
        ],
    )
    def sc_gather(w_hbm, paths_hbm, out_hbm, table_v, idx_v, out_v):
        wid = lax.axis_index("s") * _NC + lax.axis_index("c")
        pltpu.sync_copy(w_hbm, table_v)
        lanes = lax.iota(jnp.int32, _LANES)
        scale = jnp.float32(1.0 / l)
        for c in range(n_chunks):
            base = (wid * n_chunks + c) * chunk_elems
            pltpu.sync_copy(paths_hbm.at[pl.ds(base * l, chunk_elems * l)], idx_v)

            @plsc.parallel_loop(0, groups, unroll=4)
            def _grp(j):
                p0 = j * (_LANES * l)
                acc = jnp.zeros((_LANES,), jnp.float32)
                for i in range(l):
                    pos = lanes * l + (p0 + i)
                    ev = plsc.load_gather(idx_v, [pos])
                    acc = acc + plsc.load_gather(table_v, [ev + i * e])
                out_v[pl.ds(j * _LANES, _LANES)] = acc * scale

            pltpu.sync_copy(out_v, out_hbm.at[pl.ds(base, chunk_elems)])

    return sc_gather


def kernel(x, edge_attr, edge_paths, edge_vector):
    n = edge_paths.shape[0]
    l, d_edge = edge_vector.shape
    e = edge_attr.shape[0]

    w = pl.pallas_call(
        _w_table_kernel,
        out_shape=jax.ShapeDtypeStruct((l, e), jnp.float32),
    )(edge_vector, edge_attr)

    nw = _NC * _NS
    n_chunks = 2
    chunk_elems = (n * n) // (nw * n_chunks)

    sc_gather = _make_sc_gather(n, e, l, chunk_elems, n_chunks)
    out_flat = sc_gather(w.reshape(l * e), edge_paths.reshape(n * n * l))
    return out_flat.reshape(n, n)


# speedup vs baseline: 153.4049x; 3.6457x over previous
import functools

import jax
import jax.numpy as jnp
from jax import lax
from jax.experimental import pallas as pl
from jax.experimental.pallas import tpu as pltpu
from jax.experimental.pallas import tpu_sc as plsc

_NC = 2
_NS = 16
_LANES = 16


def _w_table_kernel(vec_ref, attr_ref, out_ref):
    out_ref[...] = lax.dot_general(
        vec_ref[...], attr_ref[...],
        (((1,), (1,)), ((), ())),
        preferred_element_type=jnp.float32,
    )


def _make_sc_gather(n, e, l, chunk_elems, n_chunks):
    mesh = plsc.VectorSubcoreMesh(core_axis_name="c", subcore_axis_name="s")
    groups = chunk_elems // _LANES
    nn = n * n

    @functools.partial(
        pl.kernel,
        out_type=jax.ShapeDtypeStruct((nn,), jnp.float32),
        mesh=mesh,
        compiler_params=pltpu.CompilerParams(needs_layout_passes=False),
        scratch_types=[
            pltpu.VMEM((l * e,), jnp.float32),
            pltpu.VMEM((l * chunk_elems,), jnp.int32),
            pltpu.VMEM((chunk_elems,), jnp.float32),
        ],
    )
    def sc_gather(w_hbm, paths_hbm, out_hbm, table_v, idx_v, out_v):
        wid = lax.axis_index("s") * _NC + lax.axis_index("c")
        pltpu.sync_copy(w_hbm, table_v)
        scale = jnp.float32(1.0 / l)
        for c in range(n_chunks):
            base = (wid * n_chunks + c) * chunk_elems
            for i in range(l):
                pltpu.sync_copy(
                    paths_hbm.at[pl.ds(i * nn + base, chunk_elems)],
                    idx_v.at[pl.ds(i * chunk_elems, chunk_elems)],
                )

            @plsc.parallel_loop(0, groups, unroll=4)
            def _grp(j):
                acc = jnp.zeros((_LANES,), jnp.float32)
                for i in range(l):
                    ev = idx_v[pl.ds(i * chunk_elems + j * _LANES, _LANES)]
                    acc = acc + plsc.load_gather(table_v, [ev + i * e])
                out_v[pl.ds(j * _LANES, _LANES)] = acc * scale

            pltpu.sync_copy(out_v, out_hbm.at[pl.ds(base, chunk_elems)])

    return sc_gather


def kernel(x, edge_attr, edge_paths, edge_vector):
    n = edge_paths.shape[0]
    l, d_edge = edge_vector.shape
    e = edge_attr.shape[0]

    w = pl.pallas_call(
        _w_table_kernel,
        out_shape=jax.ShapeDtypeStruct((l, e), jnp.float32),
    )(edge_vector, edge_attr)

    nw = _NC * _NS
    n_chunks = 2
    chunk_elems = (n * n) // (nw * n_chunks)

    paths_hm = edge_paths.transpose(2, 0, 1).reshape(l * n * n)
    sc_gather = _make_sc_gather(n, e, l, chunk_elems, n_chunks)
    out_flat = sc_gather(w.reshape(l * e), paths_hm)
    return out_flat.reshape(n, n)


# speedup vs baseline: 204.1855x; 1.3310x over previous
import functools

import jax
import jax.numpy as jnp
from jax import lax
from jax.experimental import pallas as pl
from jax.experimental.pallas import tpu as pltpu
from jax.experimental.pallas import tpu_sc as plsc

_NC = 2
_NS = 16
_LANES = 16
_BLOCKS = 4


def _w_table_kernel(vec_ref, attr_t_ref, out_ref):
    out_ref[...] = lax.dot_general(
        vec_ref[...], attr_t_ref[...],
        (((1,), (0,)), ((), ())),
        preferred_element_type=jnp.float32,
    )


def _make_sc_gather(n, e, l, slab, n_blocks):
    mesh = plsc.VectorSubcoreMesh(core_axis_name="c", subcore_axis_name="s")
    nn = n * n
    blk = slab // n_blocks
    groups = blk // _LANES

    @functools.partial(
        pl.kernel,
        out_type=jax.ShapeDtypeStruct((nn,), jnp.float32),
        mesh=mesh,
        compiler_params=pltpu.CompilerParams(needs_layout_passes=False),
        scratch_types=[
            pltpu.VMEM((l * e,), jnp.float32),
            pltpu.VMEM((l * slab,), jnp.int32),
            pltpu.VMEM((blk,), jnp.float32),
            pltpu.VMEM((blk,), jnp.float32),
            pltpu.SemaphoreType.DMA,
            pltpu.SemaphoreType.DMA,
            pltpu.SemaphoreType.DMA,
            pltpu.SemaphoreType.DMA,
        ],
    )
    def sc_gather(w_hbm, paths_hbm, out_hbm, table_v, idx_v, out0_v, out1_v,
                  sem_t, sem_i, sem_s0, sem_s1):
        wid = lax.axis_index("s") * _NC + lax.axis_index("c")
        base = wid * slab
        in_dmas = [pltpu.async_copy(w_hbm, table_v, sem_t)]
        for i in range(l):
            in_dmas.append(pltpu.async_copy(
                paths_hbm.at[pl.ds(i * nn + base, slab)],
                idx_v.at[pl.ds(i * slab, slab)],
                sem_i,
            ))
        for dma in in_dmas:
            dma.wait()

        scale = jnp.float32(1.0 / l)
        out_bufs = (out0_v, out1_v)
        store_sems = (sem_s0, sem_s1)
        store_dmas = [None, None]
        for h in range(n_blocks):
            buf = h % 2
            if store_dmas[buf] is not None:
                store_dmas[buf].wait()

            out_v = out_bufs[buf]

            @plsc.parallel_loop(0, groups, unroll=4)
            def _grp(j):
                acc = jnp.zeros((_LANES,), jnp.float32)
                for i in range(l):
                    ev = idx_v[pl.ds(i * slab + h * blk + j * _LANES, _LANES)]
                    acc = acc + plsc.load_gather(table_v, [ev + i * e])
                out_v[pl.ds(j * _LANES, _LANES)] = acc * scale

            store_dmas[buf] = pltpu.async_copy(
                out_v, out_hbm.at[pl.ds(base + h * blk, blk)], store_sems[buf])
        for dma in store_dmas:
            if dma is not None:
                dma.wait()

    return sc_gather


def kernel(x, edge_attr, edge_paths, edge_vector):
    n = edge_paths.shape[0]
    l, d_edge = edge_vector.shape
    e = edge_attr.shape[0]

    w = pl.pallas_call(
        _w_table_kernel,
        out_shape=jax.ShapeDtypeStruct((l, e), jnp.float32),
    )(edge_vector, edge_attr.T)

    nw = _NC * _NS
    slab = (n * n) // nw

    paths_hm = edge_paths.transpose(2, 0, 1).reshape(l * n * n)
    sc_gather = _make_sc_gather(n, e, l, slab, _BLOCKS)
    out_flat = sc_gather(w.reshape(l * e), paths_hm)
    return out_flat.reshape(n, n)
